# baseline (speedup 1.0000x reference)
import jax
import jax.numpy as jnp
from jax import lax
from jax.experimental import pallas as pl
from jax.experimental.pallas import tpu as pltpu
from jax.experimental.pallas import tpu_sc as plsc

LANES = 16
LATENT = 32
NUM_WORKERS = 32
B_PER_W = 512
CHUNK = 128
NCHUNK = B_PER_W // CHUNK
NCOLS = 7813
FULL_COLS = 7812
TAIL = 1000000 - FULL_COLS * 128


def _detile_body(eu_t, ei_t, outu, outi, sem):
    wid = lax.axis_index("s") * 2 + lax.axis_index("c")

    nj = (NCOLS - wid + NUM_WORKERS - 1) // NUM_WORKERS

    def fire(k, carry):
        j = wid + k * NUM_WORKERS
        c0 = pl.multiple_of(j * 128, 128)
        pltpu.async_copy(eu_t.at[:, pl.ds(c0, 128)], outu.at[j], sem)
        pltpu.async_copy(ei_t.at[:, pl.ds(c0, 128)], outi.at[j], sem)
        return carry

    lax.fori_loop(0, nj, fire, 0)

    def drain(k, carry):
        pltpu.make_async_copy(eu_t.at[:, pl.ds(0, 128)], outu.at[0], sem).wait()
        pltpu.make_async_copy(ei_t.at[:, pl.ds(0, 128)], outi.at[0], sem).wait()
        return carry

    lax.fori_loop(0, nj, drain, 0)


def _gather_body(uid_hbm, iid_hbm, wu, wi, out_hbm,
                 uidx_v, iidx_v, idxu, idxi, eu_l, ei_l,
                 du, di, acc_v, out_v, sem):
    wid = lax.axis_index("s") * 2 + lax.axis_index("c")
    base = wid * B_PER_W

    pltpu.sync_copy(uid_hbm.at[pl.ds(base, B_PER_W)], uidx_v)
    pltpu.sync_copy(iid_hbm.at[pl.ds(base, B_PER_W)], iidx_v)

    lane_iota = lax.iota(jnp.int32, LANES)

    def do_chunk(ch, carry):
        c0 = ch * CHUNK
        for g in range(CHUNK // LANES):
            sl = pl.ds(c0 + g * LANES, LANES)
            gsl = pl.ds(g * LANES, LANES)
            ru = uidx_v[sl]
            ri = iidx_v[sl]
            bu = (ru >> 7) * 512 + ((ru & 127) >> 3)
            bi = (ri >> 7) * 512 + ((ri & 127) >> 3)
            idxu[gsl] = bu
            idxi[gsl] = bi
            eu_l[gsl] = ru & 7
            ei_l[gsl] = ri & 7

        def expand(d, carry2):
            off = d * LANES
            for g in range(CHUNK // LANES):
                gsl = pl.ds(g * LANES, LANES)
                dsl = pl.ds(d * CHUNK + g * LANES, LANES)
                idxu[dsl] = idxu[gsl] + off
                idxi[dsl] = idxi[gsl] + off
            return carry2

        lax.fori_loop(0, LATENT, expand, 0, unroll=4)

        pltpu.async_copy(wu.at[idxu], du, sem)
        pltpu.async_copy(wi.at[idxi], di, sem)
        pltpu.make_async_copy(wu.at[idxu], du, sem).wait()
        pltpu.make_async_copy(wi.at[idxi], di, sem).wait()

        for g in range(CHUNK // LANES):
            gsl = pl.ds(g * LANES, LANES)
            acc_v[gsl] = jnp.zeros((LANES,), jnp.float32)

        def dot(d, carry2):
            for g in range(CHUNK // LANES):
                gsl = pl.ds(g * LANES, LANES)
                rows = d * CHUNK + g * LANES + lane_iota
                uv = plsc.load_gather(du, [rows, eu_l[gsl]])
                iv = plsc.load_gather(di, [rows, ei_l[gsl]])
                acc_v[gsl] = acc_v[gsl] + uv * iv
            return carry2

        lax.fori_loop(0, LATENT, dot, 0, unroll=4)

        for g in range(CHUNK // LANES):
            gsl = pl.ds(g * LANES, LANES)
            out_v[pl.ds(c0 + g * LANES, LANES)] = acc_v[gsl]
        return carry

    lax.fori_loop(0, NCHUNK, do_chunk, 0)

    pltpu.sync_copy(out_v, out_hbm.at[pl.ds(base, B_PER_W)])


def kernel(user_id, item_id, emb_user, emb_item):
    batch = user_id.shape[0]
    uid = user_id.astype(jnp.int32)
    iid = item_id.astype(jnp.int32)
    eu_t = emb_user.T
    ei_t = emb_item.T

    mesh = plsc.VectorSubcoreMesh(core_axis_name="c", subcore_axis_name="s")

    detile = pl.kernel(
        _detile_body,
        out_type=(
            jax.ShapeDtypeStruct((NCOLS, LATENT, 128), jnp.float32),
            jax.ShapeDtypeStruct((NCOLS, LATENT, 128), jnp.float32),
        ),
        mesh=mesh,
        compiler_params=pltpu.CompilerParams(
            use_tc_tiling_on_sc=True, disable_bounds_checks=True),
        scratch_types=[pltpu.SemaphoreType.DMA],
    )
    outu, outi = detile(eu_t, ei_t)
    wu = outu.reshape(NCOLS * LATENT * 16, 8)
    wi = outi.reshape(NCOLS * LATENT * 16, 8)

    gather = pl.kernel(
        _gather_body,
        out_type=jax.ShapeDtypeStruct((batch,), jnp.float32),
        mesh=mesh,
        compiler_params=pltpu.CompilerParams(
            use_tc_tiling_on_sc=False, needs_layout_passes=False),
        scratch_types=[
            pltpu.VMEM((B_PER_W,), jnp.int32),
            pltpu.VMEM((B_PER_W,), jnp.int32),
            pltpu.VMEM((LATENT * CHUNK,), jnp.int32),
            pltpu.VMEM((LATENT * CHUNK,), jnp.int32),
            pltpu.VMEM((CHUNK,), jnp.int32),
            pltpu.VMEM((CHUNK,), jnp.int32),
            pltpu.VMEM((LATENT * CHUNK, 8), jnp.float32),
            pltpu.VMEM((LATENT * CHUNK, 8), jnp.float32),
            pltpu.VMEM((CHUNK,), jnp.float32),
            pltpu.VMEM((B_PER_W,), jnp.float32),
            pltpu.SemaphoreType.DMA,
        ],
    )
    out = gather(uid, iid, wu, wi)
    return out.reshape(batch, 1)


# speedup vs baseline: 1.0014x; 1.0014x over previous
import jax
import jax.numpy as jnp
from jax import lax
from jax.experimental import pallas as pl
from jax.experimental.pallas import tpu as pltpu
from jax.experimental.pallas import tpu_sc as plsc

LANES = 16
LATENT = 32
NUM_WORKERS = 32
B_PER_W = 512
CHUNK = 128
NCHUNK = B_PER_W // CHUNK
NCOLS = 7813
FULL_COLS = 7812
TAIL = 1000000 - FULL_COLS * 128


MAIN = FULL_COLS * 128
PLANE = NCOLS * 128


def _detile_body(eu4, ei4, outu, outi, sem):
    wid = lax.axis_index("s") * 2 + lax.axis_index("c")
    t = wid // 16
    p = wid % 16
    tail0 = pl.multiple_of((wid // NUM_WORKERS + FULL_COLS) * 128, 128)

    for tv, src, dst in ((0, eu4, outu), (1, ei4, outi)):
        @pl.when(t == tv)
        def _copy():
            for k in range(2):
                m = p * 2 + k
                i = m // 8
                s = m % 8
                pltpu.async_copy(src.at[i, s, pl.ds(0, MAIN)],
                                 dst.at[pl.ds(m * PLANE, MAIN)], sem)
                pltpu.async_copy(src.at[i, s, pl.ds(tail0, 128)],
                                 dst.at[pl.ds(m * PLANE + MAIN, 128)], sem)
            for k in range(2):
                pltpu.make_async_copy(src.at[0, 0, pl.ds(0, MAIN)],
                                      dst.at[pl.ds(0, MAIN)], sem).wait()
                pltpu.make_async_copy(src.at[0, 0, pl.ds(0, 128)],
                                      dst.at[pl.ds(0, 128)], sem).wait()


def _gather_body(uid_hbm, iid_hbm, wu, wi, out_hbm,
                 uidx_v, iidx_v, idxu, idxi, eu_l, ei_l,
                 du, di, acc_v, out_v, sem):
    wid = lax.axis_index("s") * 2 + lax.axis_index("c")
    base = wid * B_PER_W

    pltpu.sync_copy(uid_hbm.at[pl.ds(base, B_PER_W)], uidx_v)
    pltpu.sync_copy(iid_hbm.at[pl.ds(base, B_PER_W)], iidx_v)

    lane_iota = lax.iota(jnp.int32, LANES)

    def do_chunk(ch, carry):
        c0 = ch * CHUNK
        for g in range(CHUNK // LANES):
            sl = pl.ds(c0 + g * LANES, LANES)
            gsl = pl.ds(g * LANES, LANES)
            ru = uidx_v[sl]
            ri = iidx_v[sl]
            bu = (ru >> 7) * 16 + ((ru & 127) >> 3)
            bi = (ri >> 7) * 16 + ((ri & 127) >> 3)
            idxu[gsl] = bu
            idxi[gsl] = bi
            eu_l[gsl] = ru & 7
            ei_l[gsl] = ri & 7

        def expand(d, carry2):
            off = d * (NCOLS * 16)
            for g in range(CHUNK // LANES):
                gsl = pl.ds(g * LANES, LANES)
                dsl = pl.ds(d * CHUNK + g * LANES, LANES)
                idxu[dsl] = idxu[gsl] + off
                idxi[dsl] = idxi[gsl] + off
            return carry2

        lax.fori_loop(0, LATENT, expand, 0, unroll=4)

        pltpu.async_copy(wu.at[idxu], du, sem)
        pltpu.async_copy(wi.at[idxi], di, sem)
        pltpu.make_async_copy(wu.at[idxu], du, sem).wait()
        pltpu.make_async_copy(wi.at[idxi], di, sem).wait()

        for g in range(CHUNK // LANES):
            gsl = pl.ds(g * LANES, LANES)
            acc_v[gsl] = jnp.zeros((LANES,), jnp.float32)

        def dot(d, carry2):
            for g in range(CHUNK // LANES):
                gsl = pl.ds(g * LANES, LANES)
                rows = d * CHUNK + g * LANES + lane_iota
                uv = plsc.load_gather(du, [rows, eu_l[gsl]])
                iv = plsc.load_gather(di, [rows, ei_l[gsl]])
                acc_v[gsl] = acc_v[gsl] + uv * iv
            return carry2

        lax.fori_loop(0, LATENT, dot, 0, unroll=4)

        for g in range(CHUNK // LANES):
            gsl = pl.ds(g * LANES, LANES)
            out_v[pl.ds(c0 + g * LANES, LANES)] = acc_v[gsl]
        return carry

    lax.fori_loop(0, NCHUNK, do_chunk, 0)

    pltpu.sync_copy(out_v, out_hbm.at[pl.ds(base, B_PER_W)])


def kernel(user_id, item_id, emb_user, emb_item):
    batch = user_id.shape[0]
    uid = user_id.astype(jnp.int32)
    iid = item_id.astype(jnp.int32)
    eu4 = emb_user.T.reshape(4, 8, 1000000)
    ei4 = emb_item.T.reshape(4, 8, 1000000)

    mesh = plsc.VectorSubcoreMesh(core_axis_name="c", subcore_axis_name="s")

    detile = pl.kernel(
        _detile_body,
        out_type=(
            jax.ShapeDtypeStruct((LATENT * NCOLS * 128,), jnp.float32),
            jax.ShapeDtypeStruct((LATENT * NCOLS * 128,), jnp.float32),
        ),
        mesh=mesh,
        compiler_params=pltpu.CompilerParams(
            use_tc_tiling_on_sc=True, disable_bounds_checks=True),
        scratch_types=[pltpu.SemaphoreType.DMA],
    )
    outu, outi = detile(eu4, ei4)
    wu = outu.reshape(NCOLS * LATENT * 16, 8)
    wi = outi.reshape(NCOLS * LATENT * 16, 8)

    gather = pl.kernel(
        _gather_body,
        out_type=jax.ShapeDtypeStruct((batch,), jnp.float32),
        mesh=mesh,
        compiler_params=pltpu.CompilerParams(
            use_tc_tiling_on_sc=False, needs_layout_passes=False),
        scratch_types=[
            pltpu.VMEM((B_PER_W,), jnp.int32),
            pltpu.VMEM((B_PER_W,), jnp.int32),
            pltpu.VMEM((LATENT * CHUNK,), jnp.int32),
            pltpu.VMEM((LATENT * CHUNK,), jnp.int32),
            pltpu.VMEM((CHUNK,), jnp.int32),
            pltpu.VMEM((CHUNK,), jnp.int32),
            pltpu.VMEM((LATENT * CHUNK, 8), jnp.float32),
            pltpu.VMEM((LATENT * CHUNK, 8), jnp.float32),
            pltpu.VMEM((CHUNK,), jnp.float32),
            pltpu.VMEM((B_PER_W,), jnp.float32),
            pltpu.SemaphoreType.DMA,
        ],
    )
    out = gather(uid, iid, wu, wi)
    return out.reshape(batch, 1)


# speedup vs baseline: 1.5258x; 1.5237x over previous
import jax
import jax.numpy as jnp
from jax import lax
from jax.experimental import pallas as pl
from jax.experimental.pallas import tpu as pltpu
from jax.experimental.pallas import tpu_sc as plsc

LANES = 16
LATENT = 32
NUM_WORKERS = 32
B_PER_W = 512
CHUNK = 128
NCHUNK = B_PER_W // CHUNK
ROWS_PER_D = 125000


def _gather_body(uid_hbm, iid_hbm, wu, wi, out_hbm,
                 uidx_v, iidx_v, idxu, idxi, eu_l, ei_l,
                 du, di, acc_v, out_v, sem):
    wid = lax.axis_index("s") * 2 + lax.axis_index("c")
    base = wid * B_PER_W

    pltpu.sync_copy(uid_hbm.at[pl.ds(base, B_PER_W)], uidx_v)
    pltpu.sync_copy(iid_hbm.at[pl.ds(base, B_PER_W)], iidx_v)

    lane_iota = lax.iota(jnp.int32, LANES)

    def do_chunk(ch, carry):
        c0 = ch * CHUNK
        for g in range(CHUNK // LANES):
            sl = pl.ds(c0 + g * LANES, LANES)
            gsl = pl.ds(g * LANES, LANES)
            ru = uidx_v[sl]
            ri = iidx_v[sl]
            idxu[gsl] = ru >> 3
            idxi[gsl] = ri >> 3
            eu_l[gsl] = ru & 7
            ei_l[gsl] = ri & 7

        def expand(d, carry2):
            off = d * ROWS_PER_D
            for g in range(CHUNK // LANES):
                gsl = pl.ds(g * LANES, LANES)
                dsl = pl.ds(d * CHUNK + g * LANES, LANES)
                idxu[dsl] = idxu[gsl] + off
                idxi[dsl] = idxi[gsl] + off
            return carry2

        lax.fori_loop(0, LATENT, expand, 0, unroll=4)

        pltpu.async_copy(wu.at[idxu], du, sem)
        pltpu.async_copy(wi.at[idxi], di, sem)
        pltpu.make_async_copy(wu.at[idxu], du, sem).wait()
        pltpu.make_async_copy(wi.at[idxi], di, sem).wait()

        for g in range(CHUNK // LANES):
            gsl = pl.ds(g * LANES, LANES)
            acc_v[gsl] = jnp.zeros((LANES,), jnp.float32)

        def dot(d, carry2):
            for g in range(CHUNK // LANES):
                gsl = pl.ds(g * LANES, LANES)
                rows = d * CHUNK + g * LANES + lane_iota
                uv = plsc.load_gather(du, [rows, eu_l[gsl]])
                iv = plsc.load_gather(di, [rows, ei_l[gsl]])
                acc_v[gsl] = acc_v[gsl] + uv * iv
            return carry2

        lax.fori_loop(0, LATENT, dot, 0, unroll=4)

        for g in range(CHUNK // LANES):
            gsl = pl.ds(g * LANES, LANES)
            out_v[pl.ds(c0 + g * LANES, LANES)] = acc_v[gsl]
        return carry

    lax.fori_loop(0, NCHUNK, do_chunk, 0)

    pltpu.sync_copy(out_v, out_hbm.at[pl.ds(base, B_PER_W)])


def kernel(user_id, item_id, emb_user, emb_item):
    batch = user_id.shape[0]
    uid = user_id.astype(jnp.int32)
    iid = item_id.astype(jnp.int32)
    wu = emb_user.T.reshape(LATENT * ROWS_PER_D, 8)
    wi = emb_item.T.reshape(LATENT * ROWS_PER_D, 8)

    mesh = plsc.VectorSubcoreMesh(core_axis_name="c", subcore_axis_name="s")
    gather = pl.kernel(
        _gather_body,
        out_type=jax.ShapeDtypeStruct((batch,), jnp.float32),
        mesh=mesh,
        compiler_params=pltpu.CompilerParams(
            use_tc_tiling_on_sc=False, needs_layout_passes=False),
        scratch_types=[
            pltpu.VMEM((B_PER_W,), jnp.int32),
            pltpu.VMEM((B_PER_W,), jnp.int32),
            pltpu.VMEM((LATENT * CHUNK,), jnp.int32),
            pltpu.VMEM((LATENT * CHUNK,), jnp.int32),
            pltpu.VMEM((CHUNK,), jnp.int32),
            pltpu.VMEM((CHUNK,), jnp.int32),
            pltpu.VMEM((LATENT * CHUNK, 8), jnp.float32),
            pltpu.VMEM((LATENT * CHUNK, 8), jnp.float32),
            pltpu.VMEM((B_PER_W,), jnp.float32),
            pltpu.VMEM((B_PER_W,), jnp.float32),
            pltpu.SemaphoreType.DMA,
        ],
    )
    out = gather(uid, iid, wu, wi)
    return out.reshape(batch, 1)


# speedup vs baseline: 8.6285x; 5.6551x over previous
import jax
import jax.numpy as jnp
from jax import lax
from jax.experimental import pallas as pl
from jax.experimental.pallas import tpu as pltpu
from jax.experimental.pallas import tpu_sc as plsc

LANES = 16
LATENT = 32
NUM_WORKERS = 32
B_PER_W = 512


def _sc_body(uid_hbm, iid_hbm, eu, ei, out_hbm,
             uidx_v, iidx_v, u_rows, i_rows, out_v, sem):
    wid = lax.axis_index("s") * 2 + lax.axis_index("c")
    base = wid * B_PER_W

    pltpu.sync_copy(uid_hbm.at[pl.ds(base, B_PER_W)], uidx_v)
    pltpu.sync_copy(iid_hbm.at[pl.ds(base, B_PER_W)], iidx_v)

    pltpu.async_copy(eu.at[uidx_v], u_rows, sem)
    pltpu.async_copy(ei.at[iidx_v], i_rows, sem)
    pltpu.make_async_copy(eu.at[uidx_v], u_rows, sem).wait()
    pltpu.make_async_copy(ei.at[iidx_v], i_rows, sem).wait()

    lane_iota = lax.iota(jnp.int32, LANES)

    def chunk(ch, carry):
        rows = ch * LANES + lane_iota
        acc = jnp.zeros((LANES,), jnp.float32)
        for d in range(LATENT):
            cols = jnp.full((LANES,), d, jnp.int32)
            uv = plsc.load_gather(u_rows, [rows, cols])
            iv = plsc.load_gather(i_rows, [rows, cols])
            acc = acc + uv * iv
        out_v[pl.ds(ch * LANES, LANES)] = acc
        return carry

    lax.fori_loop(0, B_PER_W // LANES, chunk, 0)

    pltpu.sync_copy(out_v, out_hbm.at[pl.ds(base, B_PER_W)])


def kernel(user_id, item_id, emb_user, emb_item):
    batch = user_id.shape[0]
    uid = user_id.astype(jnp.int32)
    iid = item_id.astype(jnp.int32)

    mesh = plsc.VectorSubcoreMesh(core_axis_name="c", subcore_axis_name="s")
    run = pl.kernel(
        _sc_body,
        out_type=jax.ShapeDtypeStruct((batch,), jnp.float32),
        mesh=mesh,
        compiler_params=pltpu.CompilerParams(
            use_tc_tiling_on_sc=False, needs_layout_passes=False),
        scratch_types=[
            pltpu.VMEM((B_PER_W,), jnp.int32),
            pltpu.VMEM((B_PER_W,), jnp.int32),
            pltpu.VMEM((B_PER_W, LATENT), jnp.float32),
            pltpu.VMEM((B_PER_W, LATENT), jnp.float32),
            pltpu.VMEM((B_PER_W,), jnp.float32),
            pltpu.SemaphoreType.DMA,
        ],
    )
    out = run(uid, iid, emb_user, emb_item)
    return out.reshape(batch, 1)
